# initial kernel scaffold (unmeasured)
import jax
import jax.numpy as jnp
from jax import lax
from jax.experimental import pallas as pl
from jax.experimental.pallas import tpu as pltpu

N_DEV = 4
E_LOCAL = 8


def kernel(x, router_W, route_idx, expert_W, shared_W):
    n_tok, d_model = x.shape
    chunk = n_tok // N_DEV
    d_ff_out = expert_W.shape[2]

    def body(x_ref, rW_ref, idx_ref, eW_ref, sW_ref, out_ref,
             send_buf, recv_buf, send_sems, recv_sems):
        my = lax.axis_index("i")

        barrier_sem = pltpu.get_barrier_semaphore()
        for j in range(1, N_DEV):
            peer = lax.rem(my + j, N_DEV)
            pl.semaphore_signal(
                barrier_sem, inc=1,
                device_id=(peer,), device_id_type=pl.DeviceIdType.MESH,
            )
        pl.semaphore_wait(barrier_sem, N_DEV - 1)

        rW = rW_ref[:, :]
        sW_bf = sW_ref[:, :].astype(jnp.bfloat16)

        def expert_partial(t):
            xs = x_ref[pl.ds(t * chunk, chunk), :]
            idx_s = idx_ref[pl.ds(t * chunk, chunk), :]
            scores = jnp.dot(xs, rW, preferred_element_type=jnp.float32)
            m = jnp.max(scores, axis=-1, keepdims=True)
            p = jnp.exp(scores - m)
            probs = p / jnp.sum(p, axis=-1, keepdims=True)
            eids = lax.broadcasted_iota(jnp.int32, scores.shape, 1)
            gate = jnp.sum(jnp.where(eids == idx_s, probs, 0.0), axis=-1,
                           keepdims=True)
            acc = jnp.zeros((chunk, d_ff_out), jnp.float32)
            for le in range(E_LOCAL):
                e_glob = my * E_LOCAL + le
                w = jnp.where(idx_s == e_glob, gate, 0.0)
                xw = (xs * w).astype(jnp.bfloat16)
                acc = acc + jnp.dot(xw, eW_ref[le].astype(jnp.bfloat16),
                                    preferred_element_type=jnp.float32)
            return acc

        sends = []
        for j in range(1, N_DEV):
            t = lax.rem(my + j, N_DEV)
            send_buf[j - 1] = expert_partial(t).astype(jnp.bfloat16)
            rdma = pltpu.make_async_remote_copy(
                src_ref=send_buf.at[j - 1],
                dst_ref=recv_buf.at[j - 1],
                send_sem=send_sems.at[j - 1],
                recv_sem=recv_sems.at[j - 1],
                device_id=(t,),
                device_id_type=pl.DeviceIdType.MESH,
            )
            rdma.start()
            sends.append(rdma)

        own = expert_partial(my)
        xs_own = x_ref[pl.ds(my * chunk, chunk), :].astype(jnp.bfloat16)
        total = own + jnp.dot(xs_own, sW_bf, preferred_element_type=jnp.float32)

        for j in range(1, N_DEV):
            recv = pltpu.make_async_remote_copy(
                src_ref=send_buf.at[0],
                dst_ref=recv_buf.at[j - 1],
                send_sem=send_sems.at[0],
                recv_sem=recv_sems.at[j - 1],
                device_id=(my,),
                device_id_type=pl.DeviceIdType.MESH,
            )
            recv.wait_recv()
            total = total + recv_buf[j - 1].astype(jnp.float32)

        out_ref[:, :] = total

        for rdma in sends:
            rdma.wait_send()

    return pl.pallas_call(
        body,
        out_shape=jax.ShapeDtypeStruct((chunk, d_ff_out), jnp.float32),
        in_specs=[pl.BlockSpec(memory_space=pltpu.VMEM)] * 5,
        out_specs=pl.BlockSpec(memory_space=pltpu.VMEM),
        scratch_shapes=[
            pltpu.VMEM((N_DEV - 1, chunk, d_ff_out), jnp.bfloat16),
            pltpu.VMEM((N_DEV - 1, chunk, d_ff_out), jnp.bfloat16),
            pltpu.SemaphoreType.DMA((N_DEV - 1,)),
            pltpu.SemaphoreType.DMA((N_DEV - 1,)),
        ],
        compiler_params=pltpu.CompilerParams(collective_id=0),
    )(x, router_W, route_idx, expert_W, shared_W)


# baseline (device time: 73059 ns/iter reference)
import jax
import jax.numpy as jnp
from jax import lax
from jax.experimental import pallas as pl
from jax.experimental.pallas import tpu as pltpu

N_DEV = 4
E_LOCAL = 8


def kernel(x, router_W, route_idx, expert_W, shared_W):
    n_tok, d_model = x.shape
    chunk = n_tok // N_DEV
    d_ff_out = expert_W.shape[2]

    def body(x_ref, rW_ref, idx_ref, eW_ref, sW_ref, out_ref,
             send_buf, recv_buf, send_sems, recv_sems):
        my = lax.axis_index("i")

        barrier_sem = pltpu.get_barrier_semaphore()
        for j in range(1, N_DEV):
            peer = lax.rem(my + j, N_DEV)
            pl.semaphore_signal(
                barrier_sem, inc=1,
                device_id=(peer,), device_id_type=pl.DeviceIdType.MESH,
            )
        pl.semaphore_wait(barrier_sem, N_DEV - 1)

        rW = rW_ref[:, :]
        sW_bf = sW_ref[:, :].astype(jnp.bfloat16)

        def expert_partial(t):
            xs = x_ref[pl.ds(t * chunk, chunk), :]
            idx_s = idx_ref[pl.ds(t * chunk, chunk), :]
            scores = jnp.dot(xs, rW, preferred_element_type=jnp.float32)
            m = jnp.max(scores, axis=-1, keepdims=True)
            p = jnp.exp(scores - m)
            probs = p / jnp.sum(p, axis=-1, keepdims=True)
            eids = lax.broadcasted_iota(jnp.int32, scores.shape, 1)
            gate = jnp.sum(jnp.where(eids == idx_s, probs, 0.0), axis=-1,
                           keepdims=True)
            acc = jnp.zeros((chunk, d_ff_out), jnp.float32)
            for le in range(E_LOCAL):
                e_glob = my * E_LOCAL + le
                w = jnp.where(idx_s == e_glob, gate, 0.0)
                xw = (xs * w).astype(jnp.bfloat16)
                acc = acc + jnp.dot(xw, eW_ref[le].astype(jnp.bfloat16),
                                    preferred_element_type=jnp.float32)
            return acc

        sends = []
        for j in range(1, N_DEV):
            t = lax.rem(my + j, N_DEV)
            send_buf[j - 1] = expert_partial(t).astype(jnp.bfloat16)
            rdma = pltpu.make_async_remote_copy(
                src_ref=send_buf.at[j - 1],
                dst_ref=recv_buf.at[j - 1],
                send_sem=send_sems.at[j - 1],
                recv_sem=recv_sems.at[j - 1],
                device_id=(t,),
                device_id_type=pl.DeviceIdType.MESH,
            )
            rdma.start()
            sends.append(rdma)

        own = expert_partial(my)
        xs_own = x_ref[pl.ds(my * chunk, chunk), :].astype(jnp.bfloat16)
        total = own + jnp.dot(xs_own, sW_bf, preferred_element_type=jnp.float32)

        for j in range(1, N_DEV):
            recv = pltpu.make_async_remote_copy(
                src_ref=send_buf.at[0],
                dst_ref=recv_buf.at[j - 1],
                send_sem=send_sems.at[0],
                recv_sem=recv_sems.at[j - 1],
                device_id=(my,),
                device_id_type=pl.DeviceIdType.MESH,
            )
            recv.wait_recv()
            total = total + recv_buf[j - 1].astype(jnp.float32)

        out_ref[:, :] = total

        for rdma in sends:
            rdma.wait_send()

    return pl.pallas_call(
        body,
        out_shape=jax.ShapeDtypeStruct((chunk, d_ff_out), jnp.float32),
        in_specs=[pl.BlockSpec(memory_space=pltpu.VMEM)] * 5,
        out_specs=pl.BlockSpec(memory_space=pltpu.VMEM),
        scratch_shapes=[
            pltpu.VMEM((N_DEV - 1, chunk, d_ff_out), jnp.bfloat16),
            pltpu.VMEM((N_DEV - 1, chunk, d_ff_out), jnp.bfloat16),
            pltpu.SemaphoreType.DMA((N_DEV - 1,)),
            pltpu.SemaphoreType.DMA((N_DEV - 1,)),
        ],
        compiler_params=pltpu.CompilerParams(
            collective_id=0,
            vmem_limit_bytes=100 * 1024 * 1024,
        ),
    )(x, router_W, route_idx, expert_W, shared_W)
